# baseline (speedup 1.0000x reference)
import jax
import jax.numpy as jnp
from jax.experimental import pallas as pl
from jax.experimental.pallas import tpu as pltpu
from jax.experimental.pallas import tpu_sc as plsc

_R = 12800
_B = 4096
_BLK_R = 8
_BLK_B = 2048


def _sc_add(x2t, epat):
    mesh = plsc.VectorSubcoreMesh(core_axis_name="core", subcore_axis_name="subcore")

    @pl.kernel(
        mesh=mesh,
        out_type=jax.ShapeDtypeStruct((_R, _B), jnp.float32),
    )
    def k(x_hbm, e_hbm, o_hbm):
        def body(x_vmem, e_vmem, o_vmem):
            for r in range(_BLK_R):
                for kk in range(8):
                    ev = e_vmem.at[r, pl.ds(16 * kk, 16)][...]
                    for g in range(_BLK_B // 128):
                        sl = pl.ds(128 * g + 16 * kk, 16)
                        o_vmem.at[r, sl][...] = x_vmem.at[r, sl][...] + ev

        pltpu.emit_pipeline(
            body,
            grid=(_R // _BLK_R, _B // _BLK_B),
            in_specs=[
                pl.BlockSpec((_BLK_R, _BLK_B), lambda i, j: (i, j)),
                pl.BlockSpec((_BLK_R, 128), lambda i, j: (i, 0)),
            ],
            out_specs=[pl.BlockSpec((_BLK_R, _BLK_B), lambda i, j: (i, j))],
            core_axis_name=("core", "subcore"),
            dimension_semantics=(pltpu.PARALLEL, pltpu.PARALLEL),
        )(x_hbm, e_hbm, o_hbm)

    return k(x2t, epat)


def kernel(x, embedding):
    b, s, d = x.shape
    x2t = jnp.transpose(x, (1, 2, 0)).reshape(s * d, b)
    epat = jnp.broadcast_to(embedding.reshape(s * d, 1), (s * d, 128))
    out2 = _sc_add(x2t, epat)
    return jnp.transpose(out2.reshape(s, d, b), (2, 0, 1))


# speedup vs baseline: 3.8162x; 3.8162x over previous
import jax
import jax.numpy as jnp
from jax.experimental import pallas as pl
from jax.experimental.pallas import tpu as pltpu
from jax.experimental.pallas import tpu_sc as plsc

_R = 12800
_B = 4096
_BLK_R = 8
_BLK_B = 2048


def _sc_add(x2t, epat):
    mesh = plsc.VectorSubcoreMesh(core_axis_name="core", subcore_axis_name="subcore")

    @pl.kernel(
        mesh=mesh,
        out_type=jax.ShapeDtypeStruct((_R, _B), jnp.float32),
    )
    def k(x_hbm, e_hbm, o_hbm):
        def body(x_vmem, e_vmem, o_vmem):
            for r in range(_BLK_R):
                evs = [e_vmem.at[r, pl.ds(16 * kk, 16)][...] for kk in range(8)]

                @plsc.parallel_loop(0, _BLK_B, step=128, unroll=2)
                def _(g, evs=evs, r=r):
                    for kk in range(8):
                        sl = pl.ds(g + 16 * kk, 16)
                        o_vmem.at[r, sl][...] = x_vmem.at[r, sl][...] + evs[kk]

        pltpu.emit_pipeline(
            body,
            grid=(_R // _BLK_R, _B // _BLK_B),
            in_specs=[
                pl.BlockSpec((_BLK_R, _BLK_B), lambda i, j: (i, j)),
                pl.BlockSpec((_BLK_R, 128), lambda i, j: (i, 0)),
            ],
            out_specs=[pl.BlockSpec((_BLK_R, _BLK_B), lambda i, j: (i, j))],
            core_axis_name=("core", "subcore"),
            dimension_semantics=(pltpu.PARALLEL, pltpu.PARALLEL),
        )(x_hbm, e_hbm, o_hbm)

    return k(x2t, epat)


def kernel(x, embedding):
    b, s, d = x.shape
    x2t = jnp.transpose(x, (1, 2, 0)).reshape(s * d, b)
    epat = jnp.broadcast_to(embedding.reshape(s * d, 1), (s * d, 128))
    out2 = _sc_add(x2t, epat)
    return jnp.transpose(out2.reshape(s, d, b), (2, 0, 1))
